# baseline (speedup 1.0000x reference)
import functools
import jax
import jax.numpy as jnp
from jax import lax
from jax.experimental import pallas as pl
from jax.experimental.pallas import tpu as pltpu, tpu_sc as plsc

D = 1024
S = 8192
B = 4
NW = 32
ROWS_PER_W = S // NW
CHUNK = 16
CHUNK_ELEMS = CHUNK * D
N_CHUNKS = ROWS_PER_W // CHUNK


def _sc_body(x_hbm, pos_hbm, out_hbm, pos_v, x_v, sem):
    cid = lax.axis_index("c")
    sid = lax.axis_index("s")
    wid = sid * 2 + cid

    def chunk_loop(t, _):
        pos_off = (wid * ROWS_PER_W + t * CHUNK) * D
        pltpu.sync_copy(pos_hbm.at[pl.ds(pos_off, CHUNK_ELEMS)], pos_v)

        def batch_loop(b, _):
            x_off = b * (S * D) + pos_off
            pltpu.sync_copy(x_hbm.at[pl.ds(x_off, CHUNK_ELEMS)], x_v)

            def add_loop(i, _):
                sl = pl.ds(i * 16, 16)
                x_v[sl] = x_v[sl] + pos_v[sl]
                return 0

            lax.fori_loop(0, CHUNK_ELEMS // 16, add_loop, 0)
            pltpu.sync_copy(x_v, out_hbm.at[pl.ds(x_off, CHUNK_ELEMS)])
            return 0

        lax.fori_loop(0, B, batch_loop, 0)
        return 0

    lax.fori_loop(0, N_CHUNKS, chunk_loop, 0)


_sc_call = pl.kernel(
    _sc_body,
    out_type=jax.ShapeDtypeStruct((B * S * D,), jnp.float32),
    mesh=plsc.VectorSubcoreMesh(core_axis_name="c", subcore_axis_name="s"),
    scratch_types=[
        pltpu.VMEM((CHUNK_ELEMS,), jnp.float32),
        pltpu.VMEM((CHUNK_ELEMS,), jnp.float32),
        pltpu.SemaphoreType.DMA,
    ],
)


def kernel(x, pos_table):
    b, s, d = x.shape
    out = _sc_call(x.reshape(-1), pos_table.reshape(-1))
    return out.reshape(b, s, d)
